# Initial kernel scaffold; baseline (speedup 1.0000x reference)
import functools

import jax
import jax.numpy as jnp
from jax import lax
from jax.experimental import pallas as pl
from jax.experimental.pallas import tpu as pltpu
from jax.experimental.pallas import tpu_sc as plsc

N = 10000
E = 320000
D = 128
H = 128
C = 26

NC = 2
NS = 16
NW = NC * NS
CH = 100
CHUNKS_PER_W = E // (NW * CH)
ROWS_PER_SUB = N // NS
ZROWS = 125

_mesh = plsc.VectorSubcoreMesh(core_axis_name="c", subcore_axis_name="s")


def _fill_const(buf, rows, cols, value):
    @pl.loop(0, rows)
    def _(i):
        @pl.loop(0, cols, step=16)
        def _(j):
            buf[i, pl.ds(j, 16)] = jnp.full((16,), value, jnp.float32)


def _sc_segment_sum(table, src2d, dst2d):
    @functools.partial(
        pl.kernel,
        out_type=jax.ShapeDtypeStruct((NC, N, D), jnp.float32),
        mesh=_mesh,
        scratch_types=[
            pltpu.VMEM((CHUNKS_PER_W, CH), jnp.int32),
            pltpu.VMEM((CHUNKS_PER_W, CH), jnp.int32),
            pltpu.VMEM((CH, D), jnp.float32),
            pltpu.VMEM((ZROWS, D), jnp.float32),
            pltpu.VMEM_SHARED((N, D), jnp.float32),
        ],
    )
    def k(table_hbm, src_hbm, dst_hbm, out_hbm, src_v, dst_v, rows_v, zbuf, acc):
        c = lax.axis_index("c")
        s = lax.axis_index("s")
        wid = c * NS + s
        base_row = s * ROWS_PER_SUB

        _fill_const(zbuf, ZROWS, D, 0.0)

        @pl.loop(0, ROWS_PER_SUB // ZROWS)
        def _(i):
            pltpu.sync_copy(zbuf, acc.at[pl.ds(base_row + i * ZROWS, ZROWS)])

        pltpu.sync_copy(src_hbm.at[pl.ds(wid * CHUNKS_PER_W, CHUNKS_PER_W)], src_v)
        pltpu.sync_copy(dst_hbm.at[pl.ds(wid * CHUNKS_PER_W, CHUNKS_PER_W)], dst_v)
        plsc.subcore_barrier()

        @pl.loop(0, CHUNKS_PER_W)
        def _(j):
            pltpu.sync_copy(table_hbm.at[src_v.at[j]], rows_v)
            pltpu.sync_copy(rows_v, acc.at[dst_v.at[j]], add=True)

        plsc.subcore_barrier()

        @pl.loop(0, ROWS_PER_SUB // ZROWS)
        def _(i):
            r = base_row + i * ZROWS
            pltpu.sync_copy(acc.at[pl.ds(r, ZROWS)], out_hbm.at[c, pl.ds(r, ZROWS)])

    return k(table, src2d, dst2d)


def _sc_degree(dst2d):
    @functools.partial(
        pl.kernel,
        out_type=jax.ShapeDtypeStruct((NC, N, 16), jnp.float32),
        mesh=_mesh,
        scratch_types=[
            pltpu.VMEM((CHUNKS_PER_W, CH), jnp.int32),
            pltpu.VMEM((CH, 16), jnp.float32),
            pltpu.VMEM((ZROWS, 16), jnp.float32),
            pltpu.VMEM_SHARED((N, 16), jnp.float32),
        ],
    )
    def k(dst_hbm, out_hbm, dst_v, ones_v, zbuf, acc):
        c = lax.axis_index("c")
        s = lax.axis_index("s")
        wid = c * NS + s
        base_row = s * ROWS_PER_SUB

        _fill_const(zbuf, ZROWS, 16, 0.0)
        _fill_const(ones_v, CH, 16, 1.0)

        @pl.loop(0, ROWS_PER_SUB // ZROWS)
        def _(i):
            pltpu.sync_copy(zbuf, acc.at[pl.ds(base_row + i * ZROWS, ZROWS)])

        pltpu.sync_copy(dst_hbm.at[pl.ds(wid * CHUNKS_PER_W, CHUNKS_PER_W)], dst_v)
        plsc.subcore_barrier()

        @pl.loop(0, CHUNKS_PER_W)
        def _(j):
            pltpu.sync_copy(ones_v, acc.at[dst_v.at[j]], add=True)

        plsc.subcore_barrier()

        @pl.loop(0, ROWS_PER_SUB // ZROWS)
        def _(i):
            r = base_row + i * ZROWS
            pltpu.sync_copy(acc.at[pl.ds(r, ZROWS)], out_hbm.at[c, pl.ds(r, ZROWS)])

    return k(dst2d)


BT = 1000
GRID = N // BT

_CONTRACT_T = (((1,), (1,)), ((), ()))


def _tc_ln_self(x, gamma, beta, W1, b1):
    def body(x_ref, g_ref, be_ref, w_ref, b_ref, h_ref, s_ref):
        xb = x_ref[...]
        mu = jnp.mean(xb, axis=1, keepdims=True)
        var = jnp.mean((xb - mu) ** 2, axis=1, keepdims=True)
        h = (xb - mu) / jnp.sqrt(var + 1e-5) * g_ref[...] + be_ref[...]
        h_ref[...] = h
        s_ref[...] = lax.dot_general(
            h, w_ref[...], _CONTRACT_T, preferred_element_type=jnp.float32
        ) + b_ref[...]

    return pl.pallas_call(
        body,
        grid=(GRID,),
        in_specs=[
            pl.BlockSpec((BT, D), lambda i: (i, 0)),
            pl.BlockSpec((1, D), lambda i: (0, 0)),
            pl.BlockSpec((1, D), lambda i: (0, 0)),
            pl.BlockSpec((H, D), lambda i: (0, 0)),
            pl.BlockSpec((1, H), lambda i: (0, 0)),
        ],
        out_specs=[
            pl.BlockSpec((BT, H), lambda i: (i, 0)),
            pl.BlockSpec((BT, H), lambda i: (i, 0)),
        ],
        out_shape=[
            jax.ShapeDtypeStruct((N, H), jnp.float32),
            jax.ShapeDtypeStruct((N, H), jnp.float32),
        ],
    )(x, gamma.reshape(1, D), beta.reshape(1, D), W1, b1.reshape(1, H))


def _tc_combine_mid(s1, a0, a1, d0, d1, Wn1, Ws2, b2):
    def body(s_ref, a0_ref, a1_ref, d0_ref, d1_ref, wn_ref, ws_ref, b_ref,
             h_ref, s2_ref):
        deg = jnp.maximum(d0_ref[...][:, :1] + d1_ref[...][:, :1], 1.0)
        hn = (a0_ref[...] + a1_ref[...]) / deg
        h1 = jnp.maximum(
            s_ref[...] + lax.dot_general(
                hn, wn_ref[...], _CONTRACT_T, preferred_element_type=jnp.float32),
            0.0)
        h_ref[...] = h1
        s2_ref[...] = lax.dot_general(
            h1, ws_ref[...], _CONTRACT_T, preferred_element_type=jnp.float32
        ) + b_ref[...]

    return pl.pallas_call(
        body,
        grid=(GRID,),
        in_specs=[
            pl.BlockSpec((BT, H), lambda i: (i, 0)),
            pl.BlockSpec((BT, H), lambda i: (i, 0)),
            pl.BlockSpec((BT, H), lambda i: (i, 0)),
            pl.BlockSpec((BT, 16), lambda i: (i, 0)),
            pl.BlockSpec((BT, 16), lambda i: (i, 0)),
            pl.BlockSpec((H, H), lambda i: (0, 0)),
            pl.BlockSpec((H, H), lambda i: (0, 0)),
            pl.BlockSpec((1, H), lambda i: (0, 0)),
        ],
        out_specs=[
            pl.BlockSpec((BT, H), lambda i: (i, 0)),
            pl.BlockSpec((BT, H), lambda i: (i, 0)),
        ],
        out_shape=[
            jax.ShapeDtypeStruct((N, H), jnp.float32),
            jax.ShapeDtypeStruct((N, H), jnp.float32),
        ],
    )(s1, a0, a1, d0, d1, Wn1, Ws2, b2.reshape(1, H))


def _tc_combine_out(s2, a0, a1, d0, d1, Wn2, Wc, bc):
    def body(s_ref, a0_ref, a1_ref, d0_ref, d1_ref, wn_ref, wc_ref, bc_ref,
             o_ref, acc_ref):
        i = pl.program_id(0)
        deg = jnp.maximum(d0_ref[...][:, :1] + d1_ref[...][:, :1], 1.0)
        hn = (a0_ref[...] + a1_ref[...]) / deg
        h2 = jnp.maximum(
            s_ref[...] + lax.dot_general(
                hn, wn_ref[...], _CONTRACT_T, preferred_element_type=jnp.float32),
            0.0)
        bs = jnp.sum(h2, axis=0, keepdims=True)

        @pl.when(i == 0)
        def _():
            acc_ref[0:1, :] = bs

        @pl.when(i > 0)
        def _():
            acc_ref[0:1, :] = acc_ref[0:1, :] + bs

        @pl.when(i == GRID - 1)
        def _():
            hg = acc_ref[0:1, :] * (1.0 / N)
            o_ref[...] = lax.dot_general(
                hg, wc_ref[...], _CONTRACT_T, preferred_element_type=jnp.float32
            ) + bc_ref[...]

    return pl.pallas_call(
        body,
        grid=(GRID,),
        in_specs=[
            pl.BlockSpec((BT, H), lambda i: (i, 0)),
            pl.BlockSpec((BT, H), lambda i: (i, 0)),
            pl.BlockSpec((BT, H), lambda i: (i, 0)),
            pl.BlockSpec((BT, 16), lambda i: (i, 0)),
            pl.BlockSpec((BT, 16), lambda i: (i, 0)),
            pl.BlockSpec((H, H), lambda i: (0, 0)),
            pl.BlockSpec((C, H), lambda i: (0, 0)),
            pl.BlockSpec((1, C), lambda i: (0, 0)),
        ],
        out_specs=pl.BlockSpec((1, C), lambda i: (0, 0)),
        out_shape=jax.ShapeDtypeStruct((1, C), jnp.float32),
        scratch_shapes=[pltpu.VMEM((8, H), jnp.float32)],
    )(s2, a0, a1, d0, d1, Wn2, Wc, bc.reshape(1, C))


def kernel(x, edge_index, ln_gamma, ln_beta, W_self1, W_neigh1, b1,
           W_self2, W_neigh2, b2, Wc, bc):
    src2d = edge_index[0].reshape(E // CH, CH)
    dst2d = edge_index[1].reshape(E // CH, CH)

    deg2 = _sc_degree(dst2d)
    h0, s1 = _tc_ln_self(x, ln_gamma, ln_beta, W_self1, b1)
    agg1 = _sc_segment_sum(h0, src2d, dst2d)
    h1, s2 = _tc_combine_mid(s1, agg1[0], agg1[1], deg2[0], deg2[1],
                             W_neigh1, W_self2, b2)
    agg2 = _sc_segment_sum(h1, src2d, dst2d)
    logits = _tc_combine_out(s2, agg2[0], agg2[1], deg2[0], deg2[1],
                             W_neigh2, Wc, bc)
    return logits


# speedup vs baseline: 5.9771x; 5.9771x over previous
import functools

import jax
import jax.numpy as jnp
from jax import lax
from jax.experimental import pallas as pl
from jax.experimental.pallas import tpu as pltpu
from jax.experimental.pallas import tpu_sc as plsc

N = 10000
E = 320000
D = 128
H = 128
C = 26
DH = D // 2

NC = 2
NS = 16
CH = 100
CHUNKS = E // (NS * CH)
STRIPE = 624
ZROWS = 208
TAIL = N - STRIPE * NS

_mesh = plsc.VectorSubcoreMesh(core_axis_name="c", subcore_axis_name="s")
_sc_params = pltpu.CompilerParams(use_tc_tiling_on_sc=False)


def _fill_const(buf, rows, cols, value):
    @pl.loop(0, rows)
    def _(i):
        @pl.loop(0, cols, step=16)
        def _(j):
            buf[i, pl.ds(j, 16)] = jnp.full((16,), value, jnp.float32)


def _sc_segment_sum(tblA, tblB, src3d, dst3d):
    @functools.partial(
        pl.kernel,
        out_type=jax.ShapeDtypeStruct((NC, N, DH), jnp.float32),
        mesh=_mesh,
        compiler_params=_sc_params,
        scratch_types=[
            pltpu.VMEM((CHUNKS, CH), jnp.int32),
            pltpu.VMEM((CHUNKS, CH), jnp.int32),
            pltpu.VMEM((CH, DH), jnp.float32),
            pltpu.VMEM((ZROWS, DH), jnp.float32),
            pltpu.VMEM_SHARED((N, DH), jnp.float32),
        ],
    )
    def k(tblA_hbm, tblB_hbm, src_hbm, dst_hbm, out_hbm,
          src_v, dst_v, rows_v, zbuf, acc):
        c = lax.axis_index("c")
        s = lax.axis_index("s")
        base_row = s * STRIPE

        _fill_const(zbuf, ZROWS, DH, 0.0)

        @pl.loop(0, STRIPE // ZROWS)
        def _(i):
            pltpu.sync_copy(zbuf, acc.at[pl.ds(base_row + i * ZROWS, ZROWS)])

        @pl.when(s == NS - 1)
        def _():
            pltpu.sync_copy(zbuf.at[pl.ds(0, TAIL)], acc.at[pl.ds(STRIPE * NS, TAIL)])

        pltpu.sync_copy(src_hbm.at[s], src_v)
        pltpu.sync_copy(dst_hbm.at[s], dst_v)
        plsc.subcore_barrier()

        @pl.when(c == 0)
        def _():
            @pl.loop(0, CHUNKS)
            def _(j):
                pltpu.sync_copy(tblA_hbm.at[src_v.at[j]], rows_v)
                pltpu.sync_copy(rows_v, acc.at[dst_v.at[j]], add=True)

        @pl.when(c == 1)
        def _():
            @pl.loop(0, CHUNKS)
            def _(j):
                pltpu.sync_copy(tblB_hbm.at[src_v.at[j]], rows_v)
                pltpu.sync_copy(rows_v, acc.at[dst_v.at[j]], add=True)

        plsc.subcore_barrier()

        @pl.loop(0, STRIPE // ZROWS)
        def _(i):
            r = base_row + i * ZROWS
            pltpu.sync_copy(acc.at[pl.ds(r, ZROWS)], out_hbm.at[c, pl.ds(r, ZROWS)])

        @pl.when(s == NS - 1)
        def _():
            r = STRIPE * NS
            pltpu.sync_copy(acc.at[pl.ds(r, TAIL)], out_hbm.at[c, pl.ds(r, TAIL)])

    return k(tblA, tblB, src3d, dst3d)


def _sc_degree(dst3d):
    HALF = CHUNKS // 2

    @functools.partial(
        pl.kernel,
        out_type=jax.ShapeDtypeStruct((NC, N, 16), jnp.float32),
        mesh=_mesh,
        compiler_params=_sc_params,
        scratch_types=[
            pltpu.VMEM((CHUNKS, CH), jnp.int32),
            pltpu.VMEM((CH, 16), jnp.float32),
            pltpu.VMEM((ZROWS, 16), jnp.float32),
            pltpu.VMEM_SHARED((N, 16), jnp.float32),
        ],
    )
    def k(dst_hbm, out_hbm, dst_v, ones_v, zbuf, acc):
        c = lax.axis_index("c")
        s = lax.axis_index("s")
        base_row = s * STRIPE

        _fill_const(zbuf, ZROWS, 16, 0.0)
        _fill_const(ones_v, CH, 16, 1.0)

        @pl.loop(0, STRIPE // ZROWS)
        def _(i):
            pltpu.sync_copy(zbuf, acc.at[pl.ds(base_row + i * ZROWS, ZROWS)])

        @pl.when(s == NS - 1)
        def _():
            pltpu.sync_copy(zbuf.at[pl.ds(0, TAIL)], acc.at[pl.ds(STRIPE * NS, TAIL)])

        pltpu.sync_copy(dst_hbm.at[s], dst_v)
        plsc.subcore_barrier()

        @pl.loop(0, HALF)
        def _(j):
            pltpu.sync_copy(ones_v, acc.at[dst_v.at[c * HALF + j]], add=True)

        plsc.subcore_barrier()

        @pl.loop(0, STRIPE // ZROWS)
        def _(i):
            r = base_row + i * ZROWS
            pltpu.sync_copy(acc.at[pl.ds(r, ZROWS)], out_hbm.at[c, pl.ds(r, ZROWS)])

        @pl.when(s == NS - 1)
        def _():
            r = STRIPE * NS
            pltpu.sync_copy(acc.at[pl.ds(r, TAIL)], out_hbm.at[c, pl.ds(r, TAIL)])

    return k(dst3d)


BT = 1000
GRID = N // BT

_CT = (((1,), (1,)), ((), ()))


def _tc_ln_self(x, gamma, beta, W1, b1):
    def body(x_ref, g_ref, be_ref, w_ref, b_ref, hA_ref, hB_ref, s_ref):
        xb = x_ref[...]
        mu = jnp.mean(xb, axis=1, keepdims=True)
        var = jnp.mean((xb - mu) ** 2, axis=1, keepdims=True)
        h = (xb - mu) / jnp.sqrt(var + 1e-5) * g_ref[...] + be_ref[...]
        hA_ref[...] = h[:, :DH]
        hB_ref[...] = h[:, DH:]
        s_ref[...] = lax.dot_general(
            h, w_ref[...], _CT, preferred_element_type=jnp.float32
        ) + b_ref[...]

    return pl.pallas_call(
        body,
        grid=(GRID,),
        in_specs=[
            pl.BlockSpec((BT, D), lambda i: (i, 0)),
            pl.BlockSpec((1, D), lambda i: (0, 0)),
            pl.BlockSpec((1, D), lambda i: (0, 0)),
            pl.BlockSpec((H, D), lambda i: (0, 0)),
            pl.BlockSpec((1, H), lambda i: (0, 0)),
        ],
        out_specs=[
            pl.BlockSpec((BT, DH), lambda i: (i, 0)),
            pl.BlockSpec((BT, DH), lambda i: (i, 0)),
            pl.BlockSpec((BT, H), lambda i: (i, 0)),
        ],
        out_shape=[
            jax.ShapeDtypeStruct((N, DH), jnp.float32),
            jax.ShapeDtypeStruct((N, DH), jnp.float32),
            jax.ShapeDtypeStruct((N, H), jnp.float32),
        ],
    )(x, gamma.reshape(1, D), beta.reshape(1, D), W1, b1.reshape(1, H))


def _tc_combine_mid(s1, aA, aB, d0, d1, WnA, WnB, Ws2, b2):
    def body(s_ref, aA_ref, aB_ref, d0_ref, d1_ref, wnA_ref, wnB_ref, ws_ref,
             b_ref, hA_ref, hB_ref, s2_ref):
        rdeg = 1.0 / jnp.maximum(d0_ref[...][:, :1] + d1_ref[...][:, :1], 1.0)
        h1 = jnp.maximum(
            s_ref[...]
            + lax.dot_general(aA_ref[...] * rdeg, wnA_ref[...], _CT,
                              preferred_element_type=jnp.float32)
            + lax.dot_general(aB_ref[...] * rdeg, wnB_ref[...], _CT,
                              preferred_element_type=jnp.float32),
            0.0)
        hA_ref[...] = h1[:, :DH]
        hB_ref[...] = h1[:, DH:]
        s2_ref[...] = lax.dot_general(
            h1, ws_ref[...], _CT, preferred_element_type=jnp.float32
        ) + b_ref[...]

    return pl.pallas_call(
        body,
        grid=(GRID,),
        in_specs=[
            pl.BlockSpec((BT, H), lambda i: (i, 0)),
            pl.BlockSpec((BT, DH), lambda i: (i, 0)),
            pl.BlockSpec((BT, DH), lambda i: (i, 0)),
            pl.BlockSpec((BT, 16), lambda i: (i, 0)),
            pl.BlockSpec((BT, 16), lambda i: (i, 0)),
            pl.BlockSpec((H, DH), lambda i: (0, 0)),
            pl.BlockSpec((H, DH), lambda i: (0, 0)),
            pl.BlockSpec((H, H), lambda i: (0, 0)),
            pl.BlockSpec((1, H), lambda i: (0, 0)),
        ],
        out_specs=[
            pl.BlockSpec((BT, DH), lambda i: (i, 0)),
            pl.BlockSpec((BT, DH), lambda i: (i, 0)),
            pl.BlockSpec((BT, H), lambda i: (i, 0)),
        ],
        out_shape=[
            jax.ShapeDtypeStruct((N, DH), jnp.float32),
            jax.ShapeDtypeStruct((N, DH), jnp.float32),
            jax.ShapeDtypeStruct((N, H), jnp.float32),
        ],
    )(s1, aA, aB, d0, d1, WnA, WnB, Ws2, b2.reshape(1, H))


def _tc_combine_out(s2, aA, aB, d0, d1, WnA, WnB, Wc, bc):
    def body(s_ref, aA_ref, aB_ref, d0_ref, d1_ref, wnA_ref, wnB_ref, wc_ref,
             bc_ref, o_ref, acc_ref):
        i = pl.program_id(0)
        rdeg = 1.0 / jnp.maximum(d0_ref[...][:, :1] + d1_ref[...][:, :1], 1.0)
        h2 = jnp.maximum(
            s_ref[...]
            + lax.dot_general(aA_ref[...] * rdeg, wnA_ref[...], _CT,
                              preferred_element_type=jnp.float32)
            + lax.dot_general(aB_ref[...] * rdeg, wnB_ref[...], _CT,
                              preferred_element_type=jnp.float32),
            0.0)
        bs = jnp.sum(h2, axis=0, keepdims=True)

        @pl.when(i == 0)
        def _():
            acc_ref[0:1, :] = bs

        @pl.when(i > 0)
        def _():
            acc_ref[0:1, :] = acc_ref[0:1, :] + bs

        @pl.when(i == GRID - 1)
        def _():
            hg = acc_ref[0:1, :] * (1.0 / N)
            o_ref[...] = lax.dot_general(
                hg, wc_ref[...], _CT, preferred_element_type=jnp.float32
            ) + bc_ref[...]

    return pl.pallas_call(
        body,
        grid=(GRID,),
        in_specs=[
            pl.BlockSpec((BT, H), lambda i: (i, 0)),
            pl.BlockSpec((BT, DH), lambda i: (i, 0)),
            pl.BlockSpec((BT, DH), lambda i: (i, 0)),
            pl.BlockSpec((BT, 16), lambda i: (i, 0)),
            pl.BlockSpec((BT, 16), lambda i: (i, 0)),
            pl.BlockSpec((H, DH), lambda i: (0, 0)),
            pl.BlockSpec((H, DH), lambda i: (0, 0)),
            pl.BlockSpec((C, H), lambda i: (0, 0)),
            pl.BlockSpec((1, C), lambda i: (0, 0)),
        ],
        out_specs=pl.BlockSpec((1, C), lambda i: (0, 0)),
        out_shape=jax.ShapeDtypeStruct((1, C), jnp.float32),
        scratch_shapes=[pltpu.VMEM((8, H), jnp.float32)],
    )(s2, aA, aB, d0, d1, WnA, WnB, Wc, bc.reshape(1, C))


def kernel(x, edge_index, ln_gamma, ln_beta, W_self1, W_neigh1, b1,
           W_self2, W_neigh2, b2, Wc, bc):
    src3d = edge_index[0].reshape(NS, CHUNKS, CH)
    dst3d = edge_index[1].reshape(NS, CHUNKS, CH)

    deg2 = _sc_degree(dst3d)
    h0A, h0B, s1 = _tc_ln_self(x, ln_gamma, ln_beta, W_self1, b1)
    agg1 = _sc_segment_sum(h0A, h0B, src3d, dst3d)
    h1A, h1B, s2 = _tc_combine_mid(s1, agg1[0], agg1[1], deg2[0], deg2[1],
                                   W_neigh1[:, :DH], W_neigh1[:, DH:],
                                   W_self2, b2)
    agg2 = _sc_segment_sum(h1A, h1B, src3d, dst3d)
    logits = _tc_combine_out(s2, agg2[0], agg2[1], deg2[0], deg2[1],
                             W_neigh2[:, :DH], W_neigh2[:, DH:], Wc, bc)
    return logits


# speedup vs baseline: 11.6007x; 1.9409x over previous
import functools

import jax
import jax.numpy as jnp
from jax import lax
from jax.experimental import pallas as pl
from jax.experimental.pallas import tpu as pltpu
from jax.experimental.pallas import tpu_sc as plsc

N = 10000
E = 320000
D = 128
H = 128
C = 26
DH = D // 2

NC = 2
NS = 16
CH = 100
CHUNKS = E // (NS * CH)
STRIPE = 624
ZROWS = 208
TAIL = N - STRIPE * NS
NBUF = 4

_mesh = plsc.VectorSubcoreMesh(core_axis_name="c", subcore_axis_name="s")
_sc_params = pltpu.CompilerParams(use_tc_tiling_on_sc=False)


def _fill_const(buf, rows, cols, value):
    @pl.loop(0, rows)
    def _(i):
        @pl.loop(0, cols, step=16)
        def _(j):
            buf[i, pl.ds(j, 16)] = jnp.full((16,), value, jnp.float32)


def _sc_segment_sum(tblA, tblB, src3d, dst3d):
    @functools.partial(
        pl.kernel,
        out_type=jax.ShapeDtypeStruct((NC, N, DH), jnp.float32),
        mesh=_mesh,
        compiler_params=_sc_params,
        scratch_types=[
            pltpu.VMEM((CHUNKS, CH), jnp.int32),
            pltpu.VMEM((CHUNKS, CH), jnp.int32),
            pltpu.VMEM((NBUF, CH, DH), jnp.float32),
            pltpu.VMEM((ZROWS, DH), jnp.float32),
            pltpu.VMEM_SHARED((N, DH), jnp.float32),
            pltpu.SemaphoreType.DMA,
            pltpu.SemaphoreType.DMA,
            pltpu.SemaphoreType.DMA,
            pltpu.SemaphoreType.DMA,
        ],
    )
    def k(tblA_hbm, tblB_hbm, src_hbm, dst_hbm, out_hbm,
          src_v, dst_v, rows_v, zbuf, acc, sem0, sem1, sem2, sem3):
        sems = (sem0, sem1, sem2, sem3)
        c = lax.axis_index("c")
        s = lax.axis_index("s")
        base_row = s * STRIPE

        _fill_const(zbuf, ZROWS, DH, 0.0)

        @pl.loop(0, STRIPE // ZROWS)
        def _(i):
            pltpu.sync_copy(zbuf, acc.at[pl.ds(base_row + i * ZROWS, ZROWS)])

        @pl.when(s == NS - 1)
        def _():
            pltpu.sync_copy(zbuf.at[pl.ds(0, TAIL)], acc.at[pl.ds(STRIPE * NS, TAIL)])

        pltpu.sync_copy(src_hbm.at[s], src_v)
        pltpu.sync_copy(dst_hbm.at[s], dst_v)
        plsc.subcore_barrier()

        def edge_loop(tbl_hbm):
            for b in range(NBUF):
                pltpu.async_copy(tbl_hbm.at[src_v.at[b]], rows_v.at[b], sems[b])

            @pl.loop(0, CHUNKS, step=NBUF)
            def _(g):
                for b in range(NBUF):
                    j = g + b
                    pltpu.make_async_copy(
                        tbl_hbm.at[src_v.at[j]], rows_v.at[b], sems[b]).wait()
                    pltpu.sync_copy(rows_v.at[b], acc.at[dst_v.at[j]], add=True)

                    @pl.when(g + b + NBUF < CHUNKS)
                    def _():
                        pltpu.async_copy(
                            tbl_hbm.at[src_v.at[g + b + NBUF]], rows_v.at[b],
                            sems[b])

        @pl.when(c == 0)
        def _():
            edge_loop(tblA_hbm)

        @pl.when(c == 1)
        def _():
            edge_loop(tblB_hbm)

        plsc.subcore_barrier()

        @pl.loop(0, STRIPE // ZROWS)
        def _(i):
            r = base_row + i * ZROWS
            pltpu.sync_copy(acc.at[pl.ds(r, ZROWS)], out_hbm.at[c, pl.ds(r, ZROWS)])

        @pl.when(s == NS - 1)
        def _():
            r = STRIPE * NS
            pltpu.sync_copy(acc.at[pl.ds(r, TAIL)], out_hbm.at[c, pl.ds(r, TAIL)])

    return k(tblA, tblB, src3d, dst3d)


def _sc_degree(dst3d):
    HALF = CHUNKS // 2

    @functools.partial(
        pl.kernel,
        out_type=jax.ShapeDtypeStruct((NC, N, 16), jnp.float32),
        mesh=_mesh,
        compiler_params=_sc_params,
        scratch_types=[
            pltpu.VMEM((CHUNKS, CH), jnp.int32),
            pltpu.VMEM((CH, 16), jnp.float32),
            pltpu.VMEM((ZROWS, 16), jnp.float32),
            pltpu.VMEM_SHARED((N, 16), jnp.float32),
        ],
    )
    def k(dst_hbm, out_hbm, dst_v, ones_v, zbuf, acc):
        c = lax.axis_index("c")
        s = lax.axis_index("s")
        base_row = s * STRIPE

        _fill_const(zbuf, ZROWS, 16, 0.0)
        _fill_const(ones_v, CH, 16, 1.0)

        @pl.loop(0, STRIPE // ZROWS)
        def _(i):
            pltpu.sync_copy(zbuf, acc.at[pl.ds(base_row + i * ZROWS, ZROWS)])

        @pl.when(s == NS - 1)
        def _():
            pltpu.sync_copy(zbuf.at[pl.ds(0, TAIL)], acc.at[pl.ds(STRIPE * NS, TAIL)])

        pltpu.sync_copy(dst_hbm.at[s], dst_v)
        plsc.subcore_barrier()

        @pl.loop(0, HALF)
        def _(j):
            pltpu.sync_copy(ones_v, acc.at[dst_v.at[c * HALF + j]], add=True)

        plsc.subcore_barrier()

        @pl.loop(0, STRIPE // ZROWS)
        def _(i):
            r = base_row + i * ZROWS
            pltpu.sync_copy(acc.at[pl.ds(r, ZROWS)], out_hbm.at[c, pl.ds(r, ZROWS)])

        @pl.when(s == NS - 1)
        def _():
            r = STRIPE * NS
            pltpu.sync_copy(acc.at[pl.ds(r, TAIL)], out_hbm.at[c, pl.ds(r, TAIL)])

    return k(dst3d)


BT = 1000
GRID = N // BT

_CT = (((1,), (1,)), ((), ()))


def _tc_ln_self(x, gamma, beta, W1, b1):
    def body(x_ref, g_ref, be_ref, w_ref, b_ref, hA_ref, hB_ref, s_ref):
        xb = x_ref[...]
        mu = jnp.mean(xb, axis=1, keepdims=True)
        var = jnp.mean((xb - mu) ** 2, axis=1, keepdims=True)
        h = (xb - mu) / jnp.sqrt(var + 1e-5) * g_ref[...] + be_ref[...]
        hA_ref[...] = h[:, :DH]
        hB_ref[...] = h[:, DH:]
        s_ref[...] = lax.dot_general(
            h, w_ref[...], _CT, preferred_element_type=jnp.float32
        ) + b_ref[...]

    return pl.pallas_call(
        body,
        grid=(GRID,),
        in_specs=[
            pl.BlockSpec((BT, D), lambda i: (i, 0)),
            pl.BlockSpec((1, D), lambda i: (0, 0)),
            pl.BlockSpec((1, D), lambda i: (0, 0)),
            pl.BlockSpec((H, D), lambda i: (0, 0)),
            pl.BlockSpec((1, H), lambda i: (0, 0)),
        ],
        out_specs=[
            pl.BlockSpec((BT, DH), lambda i: (i, 0)),
            pl.BlockSpec((BT, DH), lambda i: (i, 0)),
            pl.BlockSpec((BT, H), lambda i: (i, 0)),
        ],
        out_shape=[
            jax.ShapeDtypeStruct((N, DH), jnp.float32),
            jax.ShapeDtypeStruct((N, DH), jnp.float32),
            jax.ShapeDtypeStruct((N, H), jnp.float32),
        ],
    )(x, gamma.reshape(1, D), beta.reshape(1, D), W1, b1.reshape(1, H))


def _tc_combine_mid(s1, aA, aB, d0, d1, WnA, WnB, Ws2, b2):
    def body(s_ref, aA_ref, aB_ref, d0_ref, d1_ref, wnA_ref, wnB_ref, ws_ref,
             b_ref, hA_ref, hB_ref, s2_ref):
        rdeg = 1.0 / jnp.maximum(d0_ref[...][:, :1] + d1_ref[...][:, :1], 1.0)
        h1 = jnp.maximum(
            s_ref[...]
            + lax.dot_general(aA_ref[...] * rdeg, wnA_ref[...], _CT,
                              preferred_element_type=jnp.float32)
            + lax.dot_general(aB_ref[...] * rdeg, wnB_ref[...], _CT,
                              preferred_element_type=jnp.float32),
            0.0)
        hA_ref[...] = h1[:, :DH]
        hB_ref[...] = h1[:, DH:]
        s2_ref[...] = lax.dot_general(
            h1, ws_ref[...], _CT, preferred_element_type=jnp.float32
        ) + b_ref[...]

    return pl.pallas_call(
        body,
        grid=(GRID,),
        in_specs=[
            pl.BlockSpec((BT, H), lambda i: (i, 0)),
            pl.BlockSpec((BT, DH), lambda i: (i, 0)),
            pl.BlockSpec((BT, DH), lambda i: (i, 0)),
            pl.BlockSpec((BT, 16), lambda i: (i, 0)),
            pl.BlockSpec((BT, 16), lambda i: (i, 0)),
            pl.BlockSpec((H, DH), lambda i: (0, 0)),
            pl.BlockSpec((H, DH), lambda i: (0, 0)),
            pl.BlockSpec((H, H), lambda i: (0, 0)),
            pl.BlockSpec((1, H), lambda i: (0, 0)),
        ],
        out_specs=[
            pl.BlockSpec((BT, DH), lambda i: (i, 0)),
            pl.BlockSpec((BT, DH), lambda i: (i, 0)),
            pl.BlockSpec((BT, H), lambda i: (i, 0)),
        ],
        out_shape=[
            jax.ShapeDtypeStruct((N, DH), jnp.float32),
            jax.ShapeDtypeStruct((N, DH), jnp.float32),
            jax.ShapeDtypeStruct((N, H), jnp.float32),
        ],
    )(s1, aA, aB, d0, d1, WnA, WnB, Ws2, b2.reshape(1, H))


def _tc_combine_out(s2, aA, aB, d0, d1, WnA, WnB, Wc, bc):
    def body(s_ref, aA_ref, aB_ref, d0_ref, d1_ref, wnA_ref, wnB_ref, wc_ref,
             bc_ref, o_ref, acc_ref):
        i = pl.program_id(0)
        rdeg = 1.0 / jnp.maximum(d0_ref[...][:, :1] + d1_ref[...][:, :1], 1.0)
        h2 = jnp.maximum(
            s_ref[...]
            + lax.dot_general(aA_ref[...] * rdeg, wnA_ref[...], _CT,
                              preferred_element_type=jnp.float32)
            + lax.dot_general(aB_ref[...] * rdeg, wnB_ref[...], _CT,
                              preferred_element_type=jnp.float32),
            0.0)
        bs = jnp.sum(h2, axis=0, keepdims=True)

        @pl.when(i == 0)
        def _():
            acc_ref[0:1, :] = bs

        @pl.when(i > 0)
        def _():
            acc_ref[0:1, :] = acc_ref[0:1, :] + bs

        @pl.when(i == GRID - 1)
        def _():
            hg = acc_ref[0:1, :] * (1.0 / N)
            o_ref[...] = lax.dot_general(
                hg, wc_ref[...], _CT, preferred_element_type=jnp.float32
            ) + bc_ref[...]

    return pl.pallas_call(
        body,
        grid=(GRID,),
        in_specs=[
            pl.BlockSpec((BT, H), lambda i: (i, 0)),
            pl.BlockSpec((BT, DH), lambda i: (i, 0)),
            pl.BlockSpec((BT, DH), lambda i: (i, 0)),
            pl.BlockSpec((BT, 16), lambda i: (i, 0)),
            pl.BlockSpec((BT, 16), lambda i: (i, 0)),
            pl.BlockSpec((H, DH), lambda i: (0, 0)),
            pl.BlockSpec((H, DH), lambda i: (0, 0)),
            pl.BlockSpec((C, H), lambda i: (0, 0)),
            pl.BlockSpec((1, C), lambda i: (0, 0)),
        ],
        out_specs=pl.BlockSpec((1, C), lambda i: (0, 0)),
        out_shape=jax.ShapeDtypeStruct((1, C), jnp.float32),
        scratch_shapes=[pltpu.VMEM((8, H), jnp.float32)],
    )(s2, aA, aB, d0, d1, WnA, WnB, Wc, bc.reshape(1, C))


def kernel(x, edge_index, ln_gamma, ln_beta, W_self1, W_neigh1, b1,
           W_self2, W_neigh2, b2, Wc, bc):
    src3d = edge_index[0].reshape(NS, CHUNKS, CH)
    dst3d = edge_index[1].reshape(NS, CHUNKS, CH)

    deg2 = _sc_degree(dst3d)
    h0A, h0B, s1 = _tc_ln_self(x, ln_gamma, ln_beta, W_self1, b1)
    agg1 = _sc_segment_sum(h0A, h0B, src3d, dst3d)
    h1A, h1B, s2 = _tc_combine_mid(s1, agg1[0], agg1[1], deg2[0], deg2[1],
                                   W_neigh1[:, :DH], W_neigh1[:, DH:],
                                   W_self2, b2)
    agg2 = _sc_segment_sum(h1A, h1B, src3d, dst3d)
    logits = _tc_combine_out(s2, agg2[0], agg2[1], deg2[0], deg2[1],
                             W_neigh2[:, :DH], W_neigh2[:, DH:], Wc, bc)
    return logits


# speedup vs baseline: 11.7628x; 1.0140x over previous
import functools

import jax
import jax.numpy as jnp
from jax import lax
from jax.experimental import pallas as pl
from jax.experimental.pallas import tpu as pltpu
from jax.experimental.pallas import tpu_sc as plsc

N = 10000
E = 320000
D = 128
H = 128
C = 26
DH = D // 2

NC = 2
NS = 16
CH = 125
CHUNKS = E // (NS * CH)
HALFW = CHUNKS // 2
STRIPE = 624
ZROWS = 208
TAIL = N - STRIPE * NS
NBUF = 4

_mesh = plsc.VectorSubcoreMesh(core_axis_name="c", subcore_axis_name="s")
_sc_params = pltpu.CompilerParams(use_tc_tiling_on_sc=False)


def _fill_const(buf, rows, cols, value):
    @pl.loop(0, rows)
    def _(i):
        @pl.loop(0, cols, step=16)
        def _(j):
            buf[i, pl.ds(j, 16)] = jnp.full((16,), value, jnp.float32)


def _sc_segment_sum(tblA, tblB, src3d, dst3d, with_deg=False):
    out_type = [jax.ShapeDtypeStruct((NC, N, DH), jnp.float32)]
    scratch = [
        pltpu.VMEM((CHUNKS, CH), jnp.int32),
        pltpu.VMEM((CHUNKS, CH), jnp.int32),
        pltpu.VMEM((NBUF, CH, DH), jnp.float32),
        pltpu.VMEM((ZROWS, DH), jnp.float32),
        pltpu.VMEM_SHARED((N, DH), jnp.float32),
    ]
    if with_deg:
        out_type.append(jax.ShapeDtypeStruct((NC, N, 16), jnp.float32))
        scratch += [
            pltpu.VMEM((CH, 16), jnp.float32),
            pltpu.VMEM((ZROWS, 16), jnp.float32),
            pltpu.VMEM_SHARED((N, 16), jnp.float32),
        ]
    scratch += [pltpu.SemaphoreType.DMA] * NBUF

    @functools.partial(
        pl.kernel,
        out_type=out_type,
        mesh=_mesh,
        compiler_params=_sc_params,
        scratch_types=scratch,
    )
    def k(tblA_hbm, tblB_hbm, src_hbm, dst_hbm, *rest):
        if with_deg:
            (out_hbm, deg_hbm, src_v, dst_v, rows_v, zbuf, acc,
             ones_v, zbuf16, dacc, *sems) = rest
        else:
            out_hbm, src_v, dst_v, rows_v, zbuf, acc, *sems = rest
        c = lax.axis_index("c")
        s = lax.axis_index("s")
        base_row = s * STRIPE

        _fill_const(zbuf, ZROWS, DH, 0.0)
        if with_deg:
            _fill_const(zbuf16, ZROWS, 16, 0.0)
            _fill_const(ones_v, CH, 16, 1.0)

        @pl.loop(0, STRIPE // ZROWS)
        def _(i):
            pltpu.sync_copy(zbuf, acc.at[pl.ds(base_row + i * ZROWS, ZROWS)])
            if with_deg:
                pltpu.sync_copy(zbuf16, dacc.at[pl.ds(base_row + i * ZROWS, ZROWS)])

        @pl.when(s == NS - 1)
        def _():
            pltpu.sync_copy(zbuf.at[pl.ds(0, TAIL)], acc.at[pl.ds(STRIPE * NS, TAIL)])
            if with_deg:
                pltpu.sync_copy(zbuf16.at[pl.ds(0, TAIL)],
                                dacc.at[pl.ds(STRIPE * NS, TAIL)])

        pltpu.sync_copy(src_hbm.at[s], src_v)
        pltpu.sync_copy(dst_hbm.at[s], dst_v)
        plsc.subcore_barrier()

        def edge_loop(tbl_hbm):
            for b in range(NBUF):
                pltpu.async_copy(tbl_hbm.at[src_v.at[b]], rows_v.at[b], sems[b])

            @pl.loop(0, CHUNKS, step=NBUF)
            def _(g):
                for b in range(NBUF):
                    j = g + b
                    pltpu.make_async_copy(
                        tbl_hbm.at[src_v.at[j]], rows_v.at[b], sems[b]).wait()
                    pltpu.sync_copy(rows_v.at[b], acc.at[dst_v.at[j]], add=True)
                    if with_deg:
                        @pl.when((j >= c * HALFW) & (j < (c + 1) * HALFW))
                        def _():
                            pltpu.sync_copy(ones_v, dacc.at[dst_v.at[j]],
                                            add=True)

                    @pl.when(g + b + NBUF < CHUNKS)
                    def _():
                        pltpu.async_copy(
                            tbl_hbm.at[src_v.at[g + b + NBUF]], rows_v.at[b],
                            sems[b])

        @pl.when(c == 0)
        def _():
            edge_loop(tblA_hbm)

        @pl.when(c == 1)
        def _():
            edge_loop(tblB_hbm)

        plsc.subcore_barrier()

        @pl.loop(0, STRIPE // ZROWS)
        def _(i):
            r = base_row + i * ZROWS
            pltpu.sync_copy(acc.at[pl.ds(r, ZROWS)], out_hbm.at[c, pl.ds(r, ZROWS)])
            if with_deg:
                pltpu.sync_copy(dacc.at[pl.ds(r, ZROWS)],
                                deg_hbm.at[c, pl.ds(r, ZROWS)])

        @pl.when(s == NS - 1)
        def _():
            r = STRIPE * NS
            pltpu.sync_copy(acc.at[pl.ds(r, TAIL)], out_hbm.at[c, pl.ds(r, TAIL)])
            if with_deg:
                pltpu.sync_copy(dacc.at[pl.ds(r, TAIL)],
                                deg_hbm.at[c, pl.ds(r, TAIL)])

    return k(tblA, tblB, src3d, dst3d)


def _sc_degree(dst3d):
    @functools.partial(
        pl.kernel,
        out_type=jax.ShapeDtypeStruct((NC, N, 16), jnp.float32),
        mesh=_mesh,
        compiler_params=_sc_params,
        scratch_types=[
            pltpu.VMEM((CHUNKS, CH), jnp.int32),
            pltpu.VMEM((CH, 16), jnp.float32),
            pltpu.VMEM((ZROWS, 16), jnp.float32),
            pltpu.VMEM_SHARED((N, 16), jnp.float32),
        ],
    )
    def k(dst_hbm, out_hbm, dst_v, ones_v, zbuf, acc):
        c = lax.axis_index("c")
        s = lax.axis_index("s")
        base_row = s * STRIPE

        _fill_const(zbuf, ZROWS, 16, 0.0)
        _fill_const(ones_v, CH, 16, 1.0)

        @pl.loop(0, STRIPE // ZROWS)
        def _(i):
            pltpu.sync_copy(zbuf, acc.at[pl.ds(base_row + i * ZROWS, ZROWS)])

        @pl.when(s == NS - 1)
        def _():
            pltpu.sync_copy(zbuf.at[pl.ds(0, TAIL)], acc.at[pl.ds(STRIPE * NS, TAIL)])

        pltpu.sync_copy(dst_hbm.at[s], dst_v)
        plsc.subcore_barrier()

        @pl.loop(0, HALFW)
        def _(j):
            pltpu.sync_copy(ones_v, acc.at[dst_v.at[c * HALFW + j]], add=True)

        plsc.subcore_barrier()

        @pl.loop(0, STRIPE // ZROWS)
        def _(i):
            r = base_row + i * ZROWS
            pltpu.sync_copy(acc.at[pl.ds(r, ZROWS)], out_hbm.at[c, pl.ds(r, ZROWS)])

        @pl.when(s == NS - 1)
        def _():
            r = STRIPE * NS
            pltpu.sync_copy(acc.at[pl.ds(r, TAIL)], out_hbm.at[c, pl.ds(r, TAIL)])

    return k(dst3d)


BT = 1000
GRID = N // BT

_CT = (((1,), (1,)), ((), ()))


def _tc_ln_self(x, gamma, beta, W1, b1):
    def body(x_ref, g_ref, be_ref, w_ref, b_ref, hA_ref, hB_ref, s_ref):
        xb = x_ref[...]
        mu = jnp.mean(xb, axis=1, keepdims=True)
        var = jnp.mean((xb - mu) ** 2, axis=1, keepdims=True)
        h = (xb - mu) / jnp.sqrt(var + 1e-5) * g_ref[...] + be_ref[...]
        hA_ref[...] = h[:, :DH]
        hB_ref[...] = h[:, DH:]
        s_ref[...] = lax.dot_general(
            h, w_ref[...], _CT, preferred_element_type=jnp.float32
        ) + b_ref[...]

    return pl.pallas_call(
        body,
        grid=(GRID,),
        in_specs=[
            pl.BlockSpec((BT, D), lambda i: (i, 0)),
            pl.BlockSpec((1, D), lambda i: (0, 0)),
            pl.BlockSpec((1, D), lambda i: (0, 0)),
            pl.BlockSpec((H, D), lambda i: (0, 0)),
            pl.BlockSpec((1, H), lambda i: (0, 0)),
        ],
        out_specs=[
            pl.BlockSpec((BT, DH), lambda i: (i, 0)),
            pl.BlockSpec((BT, DH), lambda i: (i, 0)),
            pl.BlockSpec((BT, H), lambda i: (i, 0)),
        ],
        out_shape=[
            jax.ShapeDtypeStruct((N, DH), jnp.float32),
            jax.ShapeDtypeStruct((N, DH), jnp.float32),
            jax.ShapeDtypeStruct((N, H), jnp.float32),
        ],
    )(x, gamma.reshape(1, D), beta.reshape(1, D), W1, b1.reshape(1, H))


def _tc_combine_mid(s1, aA, aB, d0, d1, WnA, WnB, Ws2, b2):
    def body(s_ref, aA_ref, aB_ref, d0_ref, d1_ref, wnA_ref, wnB_ref, ws_ref,
             b_ref, hA_ref, hB_ref, s2_ref):
        rdeg = 1.0 / jnp.maximum(d0_ref[...][:, :1] + d1_ref[...][:, :1], 1.0)
        h1 = jnp.maximum(
            s_ref[...]
            + lax.dot_general(aA_ref[...] * rdeg, wnA_ref[...], _CT,
                              preferred_element_type=jnp.float32)
            + lax.dot_general(aB_ref[...] * rdeg, wnB_ref[...], _CT,
                              preferred_element_type=jnp.float32),
            0.0)
        hA_ref[...] = h1[:, :DH]
        hB_ref[...] = h1[:, DH:]
        s2_ref[...] = lax.dot_general(
            h1, ws_ref[...], _CT, preferred_element_type=jnp.float32
        ) + b_ref[...]

    return pl.pallas_call(
        body,
        grid=(GRID,),
        in_specs=[
            pl.BlockSpec((BT, H), lambda i: (i, 0)),
            pl.BlockSpec((BT, DH), lambda i: (i, 0)),
            pl.BlockSpec((BT, DH), lambda i: (i, 0)),
            pl.BlockSpec((BT, 16), lambda i: (i, 0)),
            pl.BlockSpec((BT, 16), lambda i: (i, 0)),
            pl.BlockSpec((H, DH), lambda i: (0, 0)),
            pl.BlockSpec((H, DH), lambda i: (0, 0)),
            pl.BlockSpec((H, H), lambda i: (0, 0)),
            pl.BlockSpec((1, H), lambda i: (0, 0)),
        ],
        out_specs=[
            pl.BlockSpec((BT, DH), lambda i: (i, 0)),
            pl.BlockSpec((BT, DH), lambda i: (i, 0)),
            pl.BlockSpec((BT, H), lambda i: (i, 0)),
        ],
        out_shape=[
            jax.ShapeDtypeStruct((N, DH), jnp.float32),
            jax.ShapeDtypeStruct((N, DH), jnp.float32),
            jax.ShapeDtypeStruct((N, H), jnp.float32),
        ],
    )(s1, aA, aB, d0, d1, WnA, WnB, Ws2, b2.reshape(1, H))


def _tc_combine_out(s2, aA, aB, d0, d1, WnA, WnB, Wc, bc):
    def body(s_ref, aA_ref, aB_ref, d0_ref, d1_ref, wnA_ref, wnB_ref, wc_ref,
             bc_ref, o_ref, acc_ref):
        i = pl.program_id(0)
        rdeg = 1.0 / jnp.maximum(d0_ref[...][:, :1] + d1_ref[...][:, :1], 1.0)
        h2 = jnp.maximum(
            s_ref[...]
            + lax.dot_general(aA_ref[...] * rdeg, wnA_ref[...], _CT,
                              preferred_element_type=jnp.float32)
            + lax.dot_general(aB_ref[...] * rdeg, wnB_ref[...], _CT,
                              preferred_element_type=jnp.float32),
            0.0)
        bs = jnp.sum(h2, axis=0, keepdims=True)

        @pl.when(i == 0)
        def _():
            acc_ref[0:1, :] = bs

        @pl.when(i > 0)
        def _():
            acc_ref[0:1, :] = acc_ref[0:1, :] + bs

        @pl.when(i == GRID - 1)
        def _():
            hg = acc_ref[0:1, :] * (1.0 / N)
            o_ref[...] = lax.dot_general(
                hg, wc_ref[...], _CT, preferred_element_type=jnp.float32
            ) + bc_ref[...]

    return pl.pallas_call(
        body,
        grid=(GRID,),
        in_specs=[
            pl.BlockSpec((BT, H), lambda i: (i, 0)),
            pl.BlockSpec((BT, DH), lambda i: (i, 0)),
            pl.BlockSpec((BT, DH), lambda i: (i, 0)),
            pl.BlockSpec((BT, 16), lambda i: (i, 0)),
            pl.BlockSpec((BT, 16), lambda i: (i, 0)),
            pl.BlockSpec((H, DH), lambda i: (0, 0)),
            pl.BlockSpec((H, DH), lambda i: (0, 0)),
            pl.BlockSpec((C, H), lambda i: (0, 0)),
            pl.BlockSpec((1, C), lambda i: (0, 0)),
        ],
        out_specs=pl.BlockSpec((1, C), lambda i: (0, 0)),
        out_shape=jax.ShapeDtypeStruct((1, C), jnp.float32),
        scratch_shapes=[pltpu.VMEM((8, H), jnp.float32)],
    )(s2, aA, aB, d0, d1, WnA, WnB, Wc, bc.reshape(1, C))


def kernel(x, edge_index, ln_gamma, ln_beta, W_self1, W_neigh1, b1,
           W_self2, W_neigh2, b2, Wc, bc):
    src3d = edge_index[0].reshape(NS, CHUNKS, CH)
    dst3d = edge_index[1].reshape(NS, CHUNKS, CH)

    deg2 = _sc_degree(dst3d)
    h0A, h0B, s1 = _tc_ln_self(x, ln_gamma, ln_beta, W_self1, b1)
    [agg1] = _sc_segment_sum(h0A, h0B, src3d, dst3d)
    h1A, h1B, s2 = _tc_combine_mid(s1, agg1[0], agg1[1], deg2[0], deg2[1],
                                   W_neigh1[:, :DH], W_neigh1[:, DH:],
                                   W_self2, b2)
    [agg2] = _sc_segment_sum(h1A, h1B, src3d, dst3d)
    logits = _tc_combine_out(s2, agg2[0], agg2[1], deg2[0], deg2[1],
                             W_neigh2[:, :DH], W_neigh2[:, DH:], Wc, bc)
    return logits
